# baseline (speedup 1.0000x reference)
import functools

import jax
import jax.numpy as jnp
from jax import lax
from jax.experimental import pallas as pl
from jax.experimental.pallas import tpu as pltpu
from jax.experimental.pallas import tpu_sc as plsc

D_MODEL = 512
MAX_MEM = 100000
BATCH = 4096
TOP_K = 16
EPS = 1e-12

CB = 2048
RB = 512
M_PAD = 100352
NCB = M_PAD // CB
NRB = BATCH // RB
VM = M_PAD // TOP_K
VCB = CB // TOP_K

NEG_INF = float("-inf")

N_WORKERS = 32
ROWS_PER_W = BATCH // N_WORKERS
CHUNKS = (256, 512, 1024, 2048, 2432)
assert sum(CHUNKS) == VM
GROUP = 2


def _sims_kernel(qk_ref, mk_ref, qid_ref, out_ref, vmax_ref):
    j = pl.program_id(1)
    s = lax.dot_general(
        qk_ref[...], mk_ref[...],
        (((1,), (1,)), ((), ())),
        preferred_element_type=jnp.float32,
    )
    col = j * CB + lax.broadcasted_iota(jnp.int32, (RB, CB), 1)
    qid = qid_ref[0, 0, :]
    bad = (col == qid[:, None]) | (col >= MAX_MEM) | (s < 0.0)
    sm = jnp.where(bad, NEG_INF, s)
    out_ref[...] = sm
    vmax_ref[...] = jnp.max(sm.reshape(RB, VCB, TOP_K), axis=2)


def _compute_sims(query_keys, mem_keys, query_ids):
    qid3 = query_ids.reshape(NRB, 1, RB)
    return pl.pallas_call(
        _sims_kernel,
        grid=(NRB, NCB),
        in_specs=[
            pl.BlockSpec((RB, D_MODEL), lambda i, j: (i, 0)),
            pl.BlockSpec((CB, D_MODEL), lambda i, j: (j, 0)),
            pl.BlockSpec((1, 1, RB), lambda i, j: (i, 0, 0)),
        ],
        out_specs=[
            pl.BlockSpec((RB, CB), lambda i, j: (i, j)),
            pl.BlockSpec((RB, VCB), lambda i, j: (i, j)),
        ],
        out_shape=[
            jax.ShapeDtypeStruct((BATCH, M_PAD), jnp.float32),
            jax.ShapeDtypeStruct((BATCH, VM), jnp.float32),
        ],
    )(query_keys, mem_keys, qid3)


def _merge_topk(top_v, top_i, cand_v, cand_i):
    cv, ci = plsc.sort_key_val(cand_v, cand_i, descending=True)
    sel = cv > top_v
    mv = jnp.where(sel, cv, top_v)
    mi = jnp.where(sel, ci, top_i)
    sv, si = plsc.sort_key_val(mv, mi, descending=False)
    return sv, si


def _topk_body(sims_ref, vmax_ref, mem_ref, ret_ref, sims_out_ref,
               ids_out_ref,
               vb0, vb1, cand, candv2, candi2, curv, idxv, g16,
               rowsa, rowsb, sims_acc, ids_acc,
               vsem0, vsem1, ssem, gsem, osem0, osem1):
    wid = lax.axis_index("s") * 2 + lax.axis_index("c")
    lane = lax.broadcasted_iota(jnp.int32, (TOP_K,), 0)

    row0 = wid * ROWS_PER_W
    pltpu.async_copy(vmax_ref.at[row0], vb0, vsem0)
    pltpu.async_copy(vmax_ref.at[row0 + 1], vb1, vsem1)

    def process_row(i, r, vbuf, vsem, rowbuf, osem):
        row = row0 + r
        pltpu.make_async_copy(vmax_ref.at[row], vbuf, vsem).wait()

        top_v = jnp.full((TOP_K,), NEG_INF, jnp.float32)
        top_i = jnp.zeros((TOP_K,), jnp.int32)
        thresh = jnp.full((TOP_K,), NEG_INF, jnp.float32)
        t_s = NEG_INF

        base = 0
        for size in CHUNKS:
            curv[...] = jnp.zeros((TOP_K,), jnp.int32)

            def filt(g, _):
                goff = base + g * (GROUP * TOP_K)
                mx = vbuf[pl.ds(goff, TOP_K)]
                for k in range(1, GROUP):
                    mx = jnp.maximum(mx, vbuf[pl.ds(goff + k * TOP_K, TOP_K)])

                @pl.when(jnp.max(mx) > t_s)
                def _hot():
                    cur = curv[...]
                    for k in range(GROUP):
                        v = vbuf[pl.ds(goff + k * TOP_K, TOP_K)]
                        m = v > thresh
                        pos = plsc.cumsum(jnp.where(m, 1, 0))
                        tgt = cur + pos - 1
                        plsc.store_scatter(
                            cand, [tgt], goff + k * TOP_K + lane, mask=m)
                        cur = cur + plsc.all_reduce_population_count(m)
                    curv[...] = cur
                return 0

            lax.fori_loop(0, size // (GROUP * TOP_K), filt, 0)

            cur = curv[...]
            n = jnp.max(cur)
            n_splat = cur

            def merge(b, carry):
                tv, ti = carry
                lidx = cand[pl.ds(b * TOP_K, TOP_K)]
                ok = (b * TOP_K + lane) < n_splat
                vals = plsc.load_gather(vbuf, [lidx], mask=ok)
                vals = jnp.where(ok, vals, NEG_INF)
                return _merge_topk(tv, ti, vals, lidx)

            nb = (n + TOP_K - 1) // TOP_K
            top_v, top_i = lax.fori_loop(0, nb, merge, (top_v, top_i))
            t_s = jnp.min(top_v)
            thresh = jnp.broadcast_to(t_s, (TOP_K,))
            base += size

        @pl.when(r + 2 < ROWS_PER_W)
        def _pf():
            pltpu.async_copy(vmax_ref.at[row + 2], vbuf, vsem)

        hs = []
        for k in range(TOP_K):
            hs.append(pltpu.async_copy(
                sims_ref.at[row, pl.ds(top_i[k] * TOP_K, TOP_K)],
                g16.at[k], ssem))
        for h in hs:
            h.wait()

        t16 = jnp.broadcast_to(t_s, (TOP_K,))
        curv[...] = jnp.zeros((TOP_K,), jnp.int32)
        for k in range(TOP_K):
            @pl.when(top_v[k] > NEG_INF)
            def _cand(k=k):
                cur = curv[...]
                v = g16[k, :]
                m = v >= t16
                pos = plsc.cumsum(jnp.where(m, 1, 0))
                tgt = cur + pos - 1
                plsc.store_scatter(candv2, [tgt], v, mask=m)
                plsc.store_scatter(
                    candi2, [tgt], top_i[k] * TOP_K + lane, mask=m)
                curv[...] = cur + plsc.all_reduce_population_count(m)

        cur2 = curv[...]
        n2 = jnp.max(cur2)
        n2_splat = cur2

        def merge2(b, carry):
            tv, ti = carry
            ok = (b * TOP_K + lane) < n2_splat
            vals = jnp.where(ok, candv2[pl.ds(b * TOP_K, TOP_K)], NEG_INF)
            idxs = candi2[pl.ds(b * TOP_K, TOP_K)]
            return _merge_topk(tv, ti, vals, idxs)

        fin0 = (jnp.full((TOP_K,), NEG_INF, jnp.float32),
                jnp.zeros((TOP_K,), jnp.int32))
        nb2 = (n2 + TOP_K - 1) // TOP_K
        fin_v, fin_i = lax.fori_loop(0, nb2, merge2, fin0)

        dv = lax.rev(fin_v, (0,))
        di = lax.rev(fin_i, (0,))
        valid = dv >= 0.0
        sv = jnp.where(valid, dv, 0.0)
        si = jnp.where(valid, di, -1)
        idxv[...] = jnp.where(valid, di, 0)

        @pl.when(i >= 1)
        def _drain():
            pltpu.make_async_copy(rowbuf, ret_ref.at[row - 2], osem).wait()

        gh = pltpu.async_copy(mem_ref.at[idxv], rowbuf, gsem)
        sims_acc[pl.ds(r * TOP_K, TOP_K)] = sv
        ids_acc[pl.ds(r * TOP_K, TOP_K)] = si
        gh.wait()
        pltpu.async_copy(rowbuf, ret_ref.at[row], osem)

    def pair_body(i, _):
        process_row(i, i * 2, vb0, vsem0, rowsa, osem0)
        process_row(i, i * 2 + 1, vb1, vsem1, rowsb, osem1)
        return 0

    lax.fori_loop(0, ROWS_PER_W // 2, pair_body, 0)
    pltpu.make_async_copy(
        rowsa, ret_ref.at[row0 + ROWS_PER_W - 2], osem0).wait()
    pltpu.make_async_copy(
        rowsb, ret_ref.at[row0 + ROWS_PER_W - 1], osem1).wait()
    pltpu.sync_copy(sims_acc, sims_out_ref.at[pl.ds(wid * ROWS_PER_W * TOP_K,
                                                    ROWS_PER_W * TOP_K)])
    pltpu.sync_copy(ids_acc, ids_out_ref.at[pl.ds(wid * ROWS_PER_W * TOP_K,
                                                  ROWS_PER_W * TOP_K)])


def _topk_sc(sims, vmax, memory_raw):
    mesh = plsc.VectorSubcoreMesh(core_axis_name="c", subcore_axis_name="s")
    f = pl.kernel(
        _topk_body,
        out_type=(
            jax.ShapeDtypeStruct((BATCH, TOP_K, D_MODEL), jnp.float32),
            jax.ShapeDtypeStruct((BATCH * TOP_K,), jnp.float32),
            jax.ShapeDtypeStruct((BATCH * TOP_K,), jnp.int32),
        ),
        mesh=mesh,
        compiler_params=pltpu.CompilerParams(needs_layout_passes=False),
        scratch_types=[
            pltpu.VMEM((VM,), jnp.float32),
            pltpu.VMEM((VM,), jnp.float32),
            pltpu.VMEM((VM,), jnp.int32),
            pltpu.VMEM((TOP_K * TOP_K,), jnp.float32),
            pltpu.VMEM((TOP_K * TOP_K,), jnp.int32),
            pltpu.VMEM((TOP_K,), jnp.int32),
            pltpu.VMEM((TOP_K,), jnp.int32),
            pltpu.VMEM((TOP_K, TOP_K), jnp.float32),
            pltpu.VMEM((TOP_K, D_MODEL), jnp.float32),
            pltpu.VMEM((TOP_K, D_MODEL), jnp.float32),
            pltpu.VMEM((ROWS_PER_W * TOP_K,), jnp.float32),
            pltpu.VMEM((ROWS_PER_W * TOP_K,), jnp.int32),
            pltpu.SemaphoreType.DMA,
            pltpu.SemaphoreType.DMA,
            pltpu.SemaphoreType.DMA,
            pltpu.SemaphoreType.DMA,
            pltpu.SemaphoreType.DMA,
            pltpu.SemaphoreType.DMA,
        ],
    )
    return f(sims, vmax, memory_raw)


def _l2norm(x):
    n = jnp.linalg.norm(x, ord=2, axis=-1, keepdims=True)
    return x / jnp.maximum(n, EPS)


def kernel(query_representations, memory_raw, query_ids, memory_ids):
    query_keys = _l2norm(query_representations)
    mem_keys = _l2norm(memory_raw)
    sims, vmax = _compute_sims(query_keys, mem_keys, query_ids)
    retrieved, sims_flat, ids_flat = _topk_sc(sims, vmax, memory_raw)
    sims_out = sims_flat.reshape(BATCH, TOP_K)
    retrieved_ids = ids_flat.reshape(BATCH, TOP_K)
    mask = retrieved_ids >= 0
    retrieved = jnp.where(mask[:, :, None], retrieved, 0.0)
    return retrieved, sims_out, mask, retrieved_ids


# speedup vs baseline: 3.0372x; 3.0372x over previous
import functools

import jax
import jax.numpy as jnp
from jax import lax
from jax.experimental import pallas as pl
from jax.experimental.pallas import tpu as pltpu
from jax.experimental.pallas import tpu_sc as plsc

D_MODEL = 512
MAX_MEM = 100000
BATCH = 4096
TOP_K = 16
EPS = 1e-12

CB = 2048
RB = 1024
M_PAD = 100352
NCB = M_PAD // CB
NRB = BATCH // RB
VM = M_PAD // TOP_K
VCB = CB // TOP_K

NEG_INF = float("-inf")

N_WORKERS = 32
ROWS_PER_W = BATCH // N_WORKERS
CHUNKS = (256, 512, 1024, 2048, 2432)
assert sum(CHUNKS) == VM
GROUP = 2


def _sims_kernel(qk_ref, mk_ref, qid_ref, out_ref, vmax_ref):
    j = pl.program_id(1)
    s = lax.dot_general(
        qk_ref[...], mk_ref[...],
        (((1,), (1,)), ((), ())),
        preferred_element_type=jnp.float32,
    )
    col = j * CB + lax.broadcasted_iota(jnp.int32, (RB, CB), 1)
    qid = qid_ref[0, 0, :]
    bad = (col == qid[:, None]) | (col >= MAX_MEM) | (s < 0.0)
    sm = jnp.where(bad, NEG_INF, s)
    out_ref[...] = sm.reshape(RB, 1, TOP_K, VCB)
    vmax_ref[...] = jnp.max(sm.reshape(RB, TOP_K, VCB), axis=1)


def _compute_sims(query_keys, mem_keys, query_ids):
    qid3 = query_ids.reshape(NRB, 1, RB)
    return pl.pallas_call(
        _sims_kernel,
        grid=(NRB, NCB),
        in_specs=[
            pl.BlockSpec((RB, D_MODEL), lambda i, j: (i, 0)),
            pl.BlockSpec((CB, D_MODEL), lambda i, j: (j, 0)),
            pl.BlockSpec((1, 1, RB), lambda i, j: (i, 0, 0)),
        ],
        out_specs=[
            pl.BlockSpec((RB, 1, TOP_K, VCB), lambda i, j: (i, j, 0, 0)),
            pl.BlockSpec((RB, VCB), lambda i, j: (i, j)),
        ],
        out_shape=[
            jax.ShapeDtypeStruct((BATCH, NCB, TOP_K, VCB), jnp.float32),
            jax.ShapeDtypeStruct((BATCH, VM), jnp.float32),
        ],
    )(query_keys, mem_keys, qid3)


def _merge_topk(top_v, top_i, cand_v, cand_i):
    cv, ci = plsc.sort_key_val(cand_v, cand_i, descending=True)
    sel = cv > top_v
    mv = jnp.where(sel, cv, top_v)
    mi = jnp.where(sel, ci, top_i)
    sv, si = plsc.sort_key_val(mv, mi, descending=False)
    return sv, si


def _topk_body(sims_ref, vmax_ref, mem_ref, ret_ref, sims_out_ref,
               ids_out_ref,
               vb0, vb1, cand, candv2, candi2, curv, idxv, g16,
               rowsa, rowsb, sims_acc, ids_acc,
               vsem0, vsem1, ssem, gsem, osem0, osem1):
    wid = lax.axis_index("s") * 2 + lax.axis_index("c")
    lane = lax.broadcasted_iota(jnp.int32, (TOP_K,), 0)

    row0 = wid * ROWS_PER_W
    pltpu.async_copy(vmax_ref.at[row0], vb0, vsem0)
    pltpu.async_copy(vmax_ref.at[row0 + 1], vb1, vsem1)

    def process_row(i, r, vbuf, vsem, rowbuf, osem):
        row = row0 + r
        pltpu.make_async_copy(vmax_ref.at[row], vbuf, vsem).wait()

        top_v = jnp.full((TOP_K,), NEG_INF, jnp.float32)
        top_i = jnp.zeros((TOP_K,), jnp.int32)
        thresh = jnp.full((TOP_K,), NEG_INF, jnp.float32)
        t_s = NEG_INF

        base = 0
        for size in CHUNKS:
            curv[...] = jnp.zeros((TOP_K,), jnp.int32)

            def filt(g, _):
                goff = base + g * (GROUP * TOP_K)
                mx = vbuf[pl.ds(goff, TOP_K)]
                for k in range(1, GROUP):
                    mx = jnp.maximum(mx, vbuf[pl.ds(goff + k * TOP_K, TOP_K)])

                @pl.when(jnp.max(mx) > t_s)
                def _hot():
                    cur = curv[...]
                    for k in range(GROUP):
                        v = vbuf[pl.ds(goff + k * TOP_K, TOP_K)]
                        m = v > thresh
                        pos = plsc.cumsum(jnp.where(m, 1, 0))
                        tgt = cur + pos - 1
                        plsc.store_scatter(
                            cand, [tgt], goff + k * TOP_K + lane, mask=m)
                        cur = cur + plsc.all_reduce_population_count(m)
                    curv[...] = cur
                return 0

            lax.fori_loop(0, size // (GROUP * TOP_K), filt, 0)

            cur = curv[...]
            n = jnp.max(cur)
            n_splat = cur

            def merge(b, carry):
                tv, ti = carry
                lidx = cand[pl.ds(b * TOP_K, TOP_K)]
                ok = (b * TOP_K + lane) < n_splat
                vals = plsc.load_gather(vbuf, [lidx], mask=ok)
                vals = jnp.where(ok, vals, NEG_INF)
                return _merge_topk(tv, ti, vals, lidx)

            nb = (n + TOP_K - 1) // TOP_K
            top_v, top_i = lax.fori_loop(0, nb, merge, (top_v, top_i))
            t_s = jnp.min(top_v)
            thresh = jnp.broadcast_to(t_s, (TOP_K,))
            base += size

        @pl.when(r + 2 < ROWS_PER_W)
        def _pf():
            pltpu.async_copy(vmax_ref.at[row + 2], vbuf, vsem)

        gb = []
        cs = []
        hs = []
        for k in range(TOP_K):
            g = top_i[k]
            b = g // VCB
            c = g - b * VCB
            gb.append(b * CB + c)
            cs.append(c)
            hs.append(pltpu.async_copy(
                sims_ref.at[row, b],
                g16.at[pl.ds(k * TOP_K, TOP_K)], ssem))
        for h in hs:
            h.wait()

        t16 = jnp.broadcast_to(t_s, (TOP_K,))
        curv[...] = jnp.zeros((TOP_K,), jnp.int32)
        for k in range(TOP_K):
            @pl.when(top_v[k] > NEG_INF)
            def _cand(k=k):
                cur = curv[...]
                v = plsc.load_gather(
                    g16, [k * TOP_K + lane, jnp.broadcast_to(cs[k],
                                                             (TOP_K,))])
                m = v >= t16
                pos = plsc.cumsum(jnp.where(m, 1, 0))
                tgt = cur + pos - 1
                plsc.store_scatter(candv2, [tgt], v, mask=m)
                plsc.store_scatter(
                    candi2, [tgt], gb[k] + lane * VCB, mask=m)
                curv[...] = cur + plsc.all_reduce_population_count(m)

        cur2 = curv[...]
        n2 = jnp.max(cur2)
        n2_splat = cur2

        def merge2(b, carry):
            tv, ti = carry
            ok = (b * TOP_K + lane) < n2_splat
            vals = jnp.where(ok, candv2[pl.ds(b * TOP_K, TOP_K)], NEG_INF)
            idxs = candi2[pl.ds(b * TOP_K, TOP_K)]
            return _merge_topk(tv, ti, vals, idxs)

        fin0 = (jnp.full((TOP_K,), NEG_INF, jnp.float32),
                jnp.zeros((TOP_K,), jnp.int32))
        nb2 = (n2 + TOP_K - 1) // TOP_K
        fin_v, fin_i = lax.fori_loop(0, nb2, merge2, fin0)

        dv = lax.rev(fin_v, (0,))
        di = lax.rev(fin_i, (0,))
        valid = dv >= 0.0
        sv = jnp.where(valid, dv, 0.0)
        si = jnp.where(valid, di, -1)
        idxv[...] = jnp.where(valid, di, 0)

        @pl.when(i >= 1)
        def _drain():
            pltpu.make_async_copy(rowbuf, ret_ref.at[row - 2], osem).wait()

        gh = pltpu.async_copy(mem_ref.at[idxv], rowbuf, gsem)
        sims_acc[pl.ds(r * TOP_K, TOP_K)] = sv
        ids_acc[pl.ds(r * TOP_K, TOP_K)] = si
        gh.wait()
        pltpu.async_copy(rowbuf, ret_ref.at[row], osem)

    def pair_body(i, _):
        process_row(i, i * 2, vb0, vsem0, rowsa, osem0)
        process_row(i, i * 2 + 1, vb1, vsem1, rowsb, osem1)
        return 0

    lax.fori_loop(0, ROWS_PER_W // 2, pair_body, 0)
    pltpu.make_async_copy(
        rowsa, ret_ref.at[row0 + ROWS_PER_W - 2], osem0).wait()
    pltpu.make_async_copy(
        rowsb, ret_ref.at[row0 + ROWS_PER_W - 1], osem1).wait()
    pltpu.sync_copy(sims_acc, sims_out_ref.at[pl.ds(wid * ROWS_PER_W * TOP_K,
                                                    ROWS_PER_W * TOP_K)])
    pltpu.sync_copy(ids_acc, ids_out_ref.at[pl.ds(wid * ROWS_PER_W * TOP_K,
                                                  ROWS_PER_W * TOP_K)])


def _topk_sc(sims, vmax, memory_raw):
    mesh = plsc.VectorSubcoreMesh(core_axis_name="c", subcore_axis_name="s")
    f = pl.kernel(
        _topk_body,
        out_type=(
            jax.ShapeDtypeStruct((BATCH, TOP_K, D_MODEL), jnp.float32),
            jax.ShapeDtypeStruct((BATCH * TOP_K,), jnp.float32),
            jax.ShapeDtypeStruct((BATCH * TOP_K,), jnp.int32),
        ),
        mesh=mesh,
        compiler_params=pltpu.CompilerParams(needs_layout_passes=False),
        scratch_types=[
            pltpu.VMEM((VM,), jnp.float32),
            pltpu.VMEM((VM,), jnp.float32),
            pltpu.VMEM((VM,), jnp.int32),
            pltpu.VMEM((TOP_K * TOP_K,), jnp.float32),
            pltpu.VMEM((TOP_K * TOP_K,), jnp.int32),
            pltpu.VMEM((TOP_K,), jnp.int32),
            pltpu.VMEM((TOP_K,), jnp.int32),
            pltpu.VMEM((TOP_K * TOP_K, VCB), jnp.float32),
            pltpu.VMEM((TOP_K, D_MODEL), jnp.float32),
            pltpu.VMEM((TOP_K, D_MODEL), jnp.float32),
            pltpu.VMEM((ROWS_PER_W * TOP_K,), jnp.float32),
            pltpu.VMEM((ROWS_PER_W * TOP_K,), jnp.int32),
            pltpu.SemaphoreType.DMA,
            pltpu.SemaphoreType.DMA,
            pltpu.SemaphoreType.DMA,
            pltpu.SemaphoreType.DMA,
            pltpu.SemaphoreType.DMA,
            pltpu.SemaphoreType.DMA,
        ],
    )
    return f(sims, vmax, memory_raw)


def _l2norm(x):
    n = jnp.linalg.norm(x, ord=2, axis=-1, keepdims=True)
    return x / jnp.maximum(n, EPS)


def kernel(query_representations, memory_raw, query_ids, memory_ids):
    query_keys = _l2norm(query_representations)
    mem_keys = _l2norm(memory_raw)
    sims, vmax = _compute_sims(query_keys, mem_keys, query_ids)
    retrieved, sims_flat, ids_flat = _topk_sc(sims, vmax, memory_raw)
    sims_out = sims_flat.reshape(BATCH, TOP_K)
    retrieved_ids = ids_flat.reshape(BATCH, TOP_K)
    mask = retrieved_ids >= 0
    retrieved = jnp.where(mask[:, :, None], retrieved, 0.0)
    return retrieved, sims_out, mask, retrieved_ids


# speedup vs baseline: 3.8625x; 1.2717x over previous
import functools

import jax
import jax.numpy as jnp
from jax import lax
from jax.experimental import pallas as pl
from jax.experimental.pallas import tpu as pltpu
from jax.experimental.pallas import tpu_sc as plsc

D_MODEL = 512
MAX_MEM = 100000
BATCH = 4096
TOP_K = 16
EPS = 1e-12

CB = 2048
RB = 1024
M_PAD = 100352
NCB = M_PAD // CB
NRB = BATCH // RB
VM = M_PAD // TOP_K
VCB = CB // TOP_K

NEG_INF = float("-inf")

SLICES = 4
SB = BATCH // SLICES
SNRB = SB // RB

N_WORKERS = 32
ROWS_PER_W = SB // N_WORKERS
CHUNKS = (256, 512, 1024, 2048, 2432)
assert sum(CHUNKS) == VM
GROUP = 2


def _sims_kernel(qk_ref, mk_ref, qid_ref, out_ref, vmax_ref):
    j = pl.program_id(1)
    s = lax.dot_general(
        qk_ref[...], mk_ref[...],
        (((1,), (1,)), ((), ())),
        preferred_element_type=jnp.float32,
    )
    col = j * CB + lax.broadcasted_iota(jnp.int32, (RB, CB), 1)
    qid = qid_ref[0, 0, :]
    bad = (col == qid[:, None]) | (col >= MAX_MEM) | (s < 0.0)
    sm = jnp.where(bad, NEG_INF, s)
    out_ref[...] = sm.reshape(RB, 1, TOP_K, VCB)
    vmax_ref[...] = jnp.max(sm.reshape(RB, TOP_K, VCB), axis=1)


def _compute_sims(query_keys, mem_keys, query_ids):
    qid3 = query_ids.reshape(SNRB, 1, RB)
    return pl.pallas_call(
        _sims_kernel,
        grid=(SNRB, NCB),
        in_specs=[
            pl.BlockSpec((RB, D_MODEL), lambda i, j: (i, 0)),
            pl.BlockSpec((CB, D_MODEL), lambda i, j: (j, 0)),
            pl.BlockSpec((1, 1, RB), lambda i, j: (i, 0, 0)),
        ],
        out_specs=[
            pl.BlockSpec((RB, 1, TOP_K, VCB), lambda i, j: (i, j, 0, 0)),
            pl.BlockSpec((RB, VCB), lambda i, j: (i, j)),
        ],
        out_shape=[
            jax.ShapeDtypeStruct((SB, NCB, TOP_K, VCB), jnp.float32),
            jax.ShapeDtypeStruct((SB, VM), jnp.float32),
        ],
    )(query_keys, mem_keys, qid3)


def _merge_topk(top_v, top_i, cand_v, cand_i):
    cv, ci = plsc.sort_key_val(cand_v, cand_i, descending=True)
    sel = cv > top_v
    mv = jnp.where(sel, cv, top_v)
    mi = jnp.where(sel, ci, top_i)
    sv, si = plsc.sort_key_val(mv, mi, descending=False)
    return sv, si


def _topk_body(sims_ref, vmax_ref, mem_ref, ret_ref, sims_out_ref,
               ids_out_ref,
               vb0, vb1, cand, candv2, candi2, curv, idxv, g16,
               rowsa, rowsb, sims_acc, ids_acc,
               vsem0, vsem1, ssem, gsem, osem0, osem1):
    wid = lax.axis_index("s") * 2 + lax.axis_index("c")
    lane = lax.broadcasted_iota(jnp.int32, (TOP_K,), 0)

    row0 = wid * ROWS_PER_W
    pltpu.async_copy(vmax_ref.at[row0], vb0, vsem0)
    pltpu.async_copy(vmax_ref.at[row0 + 1], vb1, vsem1)

    def process_row(i, r, vbuf, vsem, rowbuf, osem):
        row = row0 + r
        pltpu.make_async_copy(vmax_ref.at[row], vbuf, vsem).wait()

        top_v = jnp.full((TOP_K,), NEG_INF, jnp.float32)
        top_i = jnp.zeros((TOP_K,), jnp.int32)
        thresh = jnp.full((TOP_K,), NEG_INF, jnp.float32)
        t_s = NEG_INF

        base = 0
        for size in CHUNKS:
            curv[...] = jnp.zeros((TOP_K,), jnp.int32)

            def filt(g, _):
                goff = base + g * (GROUP * TOP_K)
                mx = vbuf[pl.ds(goff, TOP_K)]
                for k in range(1, GROUP):
                    mx = jnp.maximum(mx, vbuf[pl.ds(goff + k * TOP_K, TOP_K)])

                @pl.when(jnp.max(mx) > t_s)
                def _hot():
                    cur = curv[...]
                    for k in range(GROUP):
                        v = vbuf[pl.ds(goff + k * TOP_K, TOP_K)]
                        m = v > thresh
                        pos = plsc.cumsum(jnp.where(m, 1, 0))
                        tgt = cur + pos - 1
                        plsc.store_scatter(
                            cand, [tgt], goff + k * TOP_K + lane, mask=m)
                        cur = cur + plsc.all_reduce_population_count(m)
                    curv[...] = cur
                return 0

            lax.fori_loop(0, size // (GROUP * TOP_K), filt, 0)

            cur = curv[...]
            n = jnp.max(cur)
            n_splat = cur

            def merge(b, carry):
                tv, ti = carry
                lidx = cand[pl.ds(b * TOP_K, TOP_K)]
                ok = (b * TOP_K + lane) < n_splat
                vals = plsc.load_gather(vbuf, [lidx], mask=ok)
                vals = jnp.where(ok, vals, NEG_INF)
                return _merge_topk(tv, ti, vals, lidx)

            nb = (n + TOP_K - 1) // TOP_K
            top_v, top_i = lax.fori_loop(0, nb, merge, (top_v, top_i))
            t_s = jnp.min(top_v)
            thresh = jnp.broadcast_to(t_s, (TOP_K,))
            base += size

        @pl.when(r + 2 < ROWS_PER_W)
        def _pf():
            pltpu.async_copy(vmax_ref.at[row + 2], vbuf, vsem)

        gb = []
        cs = []
        hs = []
        for k in range(TOP_K):
            g = top_i[k]
            b = g // VCB
            c = g - b * VCB
            gb.append(b * CB + c)
            cs.append(c)
            hs.append(pltpu.async_copy(
                sims_ref.at[row, b],
                g16.at[pl.ds(k * TOP_K, TOP_K)], ssem))
        for h in hs:
            h.wait()

        t16 = jnp.broadcast_to(t_s, (TOP_K,))
        curv[...] = jnp.zeros((TOP_K,), jnp.int32)
        for k in range(TOP_K):
            @pl.when(top_v[k] > NEG_INF)
            def _cand(k=k):
                cur = curv[...]
                v = plsc.load_gather(
                    g16, [k * TOP_K + lane, jnp.broadcast_to(cs[k],
                                                             (TOP_K,))])
                m = v >= t16
                pos = plsc.cumsum(jnp.where(m, 1, 0))
                tgt = cur + pos - 1
                plsc.store_scatter(candv2, [tgt], v, mask=m)
                plsc.store_scatter(
                    candi2, [tgt], gb[k] + lane * VCB, mask=m)
                curv[...] = cur + plsc.all_reduce_population_count(m)

        cur2 = curv[...]
        n2 = jnp.max(cur2)
        n2_splat = cur2

        def merge2(b, carry):
            tv, ti = carry
            ok = (b * TOP_K + lane) < n2_splat
            vals = jnp.where(ok, candv2[pl.ds(b * TOP_K, TOP_K)], NEG_INF)
            idxs = candi2[pl.ds(b * TOP_K, TOP_K)]
            return _merge_topk(tv, ti, vals, idxs)

        fin0 = (jnp.full((TOP_K,), NEG_INF, jnp.float32),
                jnp.zeros((TOP_K,), jnp.int32))
        nb2 = (n2 + TOP_K - 1) // TOP_K
        fin_v, fin_i = lax.fori_loop(0, nb2, merge2, fin0)

        dv = lax.rev(fin_v, (0,))
        di = lax.rev(fin_i, (0,))
        valid = dv >= 0.0
        sv = jnp.where(valid, dv, 0.0)
        si = jnp.where(valid, di, -1)
        idxv[...] = jnp.where(valid, di, 0)

        @pl.when(i >= 1)
        def _drain():
            pltpu.make_async_copy(rowbuf, ret_ref.at[row - 2], osem).wait()

        gh = pltpu.async_copy(mem_ref.at[idxv], rowbuf, gsem)
        sims_acc[pl.ds(r * TOP_K, TOP_K)] = sv
        ids_acc[pl.ds(r * TOP_K, TOP_K)] = si
        gh.wait()
        pltpu.async_copy(rowbuf, ret_ref.at[row], osem)

    def pair_body(i, _):
        process_row(i, i * 2, vb0, vsem0, rowsa, osem0)
        process_row(i, i * 2 + 1, vb1, vsem1, rowsb, osem1)
        return 0

    lax.fori_loop(0, ROWS_PER_W // 2, pair_body, 0)
    pltpu.make_async_copy(
        rowsa, ret_ref.at[row0 + ROWS_PER_W - 2], osem0).wait()
    pltpu.make_async_copy(
        rowsb, ret_ref.at[row0 + ROWS_PER_W - 1], osem1).wait()
    pltpu.sync_copy(sims_acc, sims_out_ref.at[pl.ds(wid * ROWS_PER_W * TOP_K,
                                                    ROWS_PER_W * TOP_K)])
    pltpu.sync_copy(ids_acc, ids_out_ref.at[pl.ds(wid * ROWS_PER_W * TOP_K,
                                                  ROWS_PER_W * TOP_K)])


def _topk_sc(sims, vmax, memory_raw):
    mesh = plsc.VectorSubcoreMesh(core_axis_name="c", subcore_axis_name="s")
    f = pl.kernel(
        _topk_body,
        out_type=(
            jax.ShapeDtypeStruct((SB, TOP_K, D_MODEL), jnp.float32),
            jax.ShapeDtypeStruct((SB * TOP_K,), jnp.float32),
            jax.ShapeDtypeStruct((SB * TOP_K,), jnp.int32),
        ),
        mesh=mesh,
        compiler_params=pltpu.CompilerParams(needs_layout_passes=False),
        scratch_types=[
            pltpu.VMEM((VM,), jnp.float32),
            pltpu.VMEM((VM,), jnp.float32),
            pltpu.VMEM((VM,), jnp.int32),
            pltpu.VMEM((TOP_K * TOP_K,), jnp.float32),
            pltpu.VMEM((TOP_K * TOP_K,), jnp.int32),
            pltpu.VMEM((TOP_K,), jnp.int32),
            pltpu.VMEM((TOP_K,), jnp.int32),
            pltpu.VMEM((TOP_K * TOP_K, VCB), jnp.float32),
            pltpu.VMEM((TOP_K, D_MODEL), jnp.float32),
            pltpu.VMEM((TOP_K, D_MODEL), jnp.float32),
            pltpu.VMEM((ROWS_PER_W * TOP_K,), jnp.float32),
            pltpu.VMEM((ROWS_PER_W * TOP_K,), jnp.int32),
            pltpu.SemaphoreType.DMA,
            pltpu.SemaphoreType.DMA,
            pltpu.SemaphoreType.DMA,
            pltpu.SemaphoreType.DMA,
            pltpu.SemaphoreType.DMA,
            pltpu.SemaphoreType.DMA,
        ],
    )
    return f(sims, vmax, memory_raw)


def _l2norm(x):
    n = jnp.linalg.norm(x, ord=2, axis=-1, keepdims=True)
    return x / jnp.maximum(n, EPS)


def kernel(query_representations, memory_raw, query_ids, memory_ids):
    query_keys = _l2norm(query_representations)
    mem_keys = _l2norm(memory_raw)
    rets, svs, ids = [], [], []
    for s in range(SLICES):
        sl = slice(s * SB, (s + 1) * SB)
        sims, vmax = _compute_sims(query_keys[sl], mem_keys, query_ids[sl])
        r, sv, iv = _topk_sc(sims, vmax, memory_raw)
        rets.append(r)
        svs.append(sv.reshape(SB, TOP_K))
        ids.append(iv.reshape(SB, TOP_K))
    retrieved = jnp.concatenate(rets, axis=0)
    sims_out = jnp.concatenate(svs, axis=0)
    retrieved_ids = jnp.concatenate(ids, axis=0)
    mask = retrieved_ids >= 0
    retrieved = jnp.where(mask[:, :, None], retrieved, 0.0)
    return retrieved, sims_out, mask, retrieved_ids
